# baseline (speedup 1.0000x reference)
import jax
import jax.numpy as jnp
from jax import lax
from jax.experimental import pallas as pl
from jax.experimental.pallas import tpu as pltpu
from jax.experimental.pallas import tpu_sc as plsc

_INFO = plsc.get_sparse_core_info()
_NC, _NS = _INFO.num_cores, _INFO.num_subcores
_NW = _NC * _NS

_N = 16384
_S = 50
_D = 32
_NF = _D // 2
_NTAB = 1024
_SP = 56
_DP = 128
_IPW = _N // _NW
_ICH = 4
_NSLAB = _IPW // _ICH
_RSLAB = _ICH * _S
_IDXCH = 800
_SPI = _IDXCH // _RSLAB
_NIDX = _IPW * _S // _IDXCH
_UNROLL = 8
_ROWW = _S * _DP
_SLABW = _SP * _DP

_OFF_SA = 0
_OFF_CA = _NTAB * _NF
_OFF_SB = 2 * _NTAB * _NF
_OFF_CB = 3 * _NTAB * _NF


def _body(t_hbm, tabs_hbm, out_hbm, tabs, idx_v, slab_v, sem_i, sem_o):
    wid = lax.axis_index("s") * _NC + lax.axis_index("c")
    base = wid * _IPW * _S
    ibase = wid * _IPW

    pltpu.sync_copy(tabs_hbm, tabs)

    def idx_load(ic, b):
        pltpu.async_copy(t_hbm.at[pl.ds(base + ic * _IDXCH, _IDXCH)],
                         idx_v[b].at[pl.ds(0, _IDXCH)], sem_i[b])

    def wait_idx(ic, b):
        pltpu.make_async_copy(t_hbm.at[pl.ds(base + ic * _IDXCH, _IDXCH)],
                              idx_v[b].at[pl.ds(0, _IDXCH)], sem_i[b]).wait()

    def store(g, b):
        row0 = (ibase + g * _ICH) * _SLABW
        for ii in range(_ICH):
            pltpu.async_copy(
                slab_v[b].at[pl.ds(ii * _SLABW, _ROWW)],
                out_hbm.at[pl.ds(row0 + ii * _SLABW, _ROWW)], sem_o[b])

    def wait_store(g, b):
        row0 = (ibase + g * _ICH) * _SLABW
        for ii in range(_ICH):
            pltpu.make_async_copy(
                slab_v[b].at[pl.ds(ii * _SLABW, _ROWW)],
                out_hbm.at[pl.ds(row0 + ii * _SLABW, _ROWW)],
                sem_o[b]).wait()

    evens = 2 * lax.iota(jnp.int32, 16)
    odds = evens + 1
    lane = lax.iota(jnp.int32, 16)

    def compute(ib, s, b):
        @plsc.parallel_loop(0, _RSLAB // _UNROLL, unroll=4)
        def _(g):
            tvec = idx_v[ib][pl.ds(s * _RSLAB + g * _UNROLL, 16)]
            a16v = (tvec >> 10) * _NF
            b16v = (tvec & 1023) * _NF
            rrv = g * _UNROLL + lane
            iiv = rrv // _S
            posv = (iiv * (_SP - _S) + rrv) * _DP
            for u in range(_UNROLL):
                a16 = a16v[u]
                b16 = b16v[u]
                s_a = tabs[pl.ds(_OFF_SA + a16, _NF)]
                c_a = tabs[pl.ds(_OFF_CA + a16, _NF)]
                s_b = tabs[pl.ds(_OFF_SB + b16, _NF)]
                c_b = tabs[pl.ds(_OFF_CB + b16, _NF)]
                sv = s_a * c_b + c_a * s_b
                cv = c_a * c_b - s_a * s_b
                pos = jnp.full((16,), posv[u], jnp.int32)
                plsc.store_scatter(slab_v[b], [pos + evens], sv)
                plsc.store_scatter(slab_v[b], [pos + odds], cv)

    idx_load(0, 0)

    def slab_step(ic, ib, s, b, first_two):
        g = ic * _SPI + s
        if not first_two:
            wait_store(g - 2, b)
        compute(ib, s, b)
        store(g, b)

    def slab_pairs(ic, ib, lo):
        @pl.loop(lo, _SPI // 2)
        def _(sp):
            for db in range(2):
                slab_step(ic, ib, sp * 2 + db, db, first_two=False)

    def idx_chunk(ic, prefetch, first=False):
        ib = ic % 2
        wait_idx(ic, ib)
        if prefetch:
            idx_load(ic + 1, (ib + 1) % 2)
        if first:
            slab_step(ic, ib, 0, 0, first_two=True)
            slab_step(ic, ib, 1, 1, first_two=True)
            slab_pairs(ic, ib, 1)
        else:
            slab_pairs(ic, ib, 0)

    idx_chunk(0, prefetch=True, first=True)

    @pl.loop(0, (_NIDX - 2) // 2)
    def _(k):
        for db in range(2):
            ic = 1 + 2 * k + db
            ib = (1 + db) % 2
            wait_idx(ic, ib)
            idx_load(ic + 1, (ib + 1) % 2)
            slab_pairs(ic, ib, 0)

    idx_chunk(_NIDX - 1, prefetch=False)

    wait_store(_NSLAB - 2, 0)
    wait_store(_NSLAB - 1, 1)


@jax.jit
def _encode(t_flat, tabs_flat):
    mesh = plsc.VectorSubcoreMesh(core_axis_name="c", subcore_axis_name="s")
    k = pl.kernel(
        _body,
        out_type=jax.ShapeDtypeStruct((_N * _SP * _DP,), jnp.float32),
        mesh=mesh,
        scratch_types=[
            pltpu.VMEM((4 * _NTAB * _NF,), jnp.float32),
            tuple(pltpu.VMEM((_IDXCH + 16,), jnp.int32) for _ in range(2)),
            tuple(pltpu.VMEM((_ICH * _SLABW,), jnp.float32)
                  for _ in range(2)),
            tuple(pltpu.SemaphoreType.DMA for _ in range(2)),
            tuple(pltpu.SemaphoreType.DMA for _ in range(2)),
        ],
        compiler_params=pltpu.CompilerParams(use_tc_tiling_on_sc=False,
                                             needs_layout_passes=False),
    )
    return k(t_flat, tabs_flat)


def kernel(t, embeddings):
    coarse = jnp.pad(embeddings[::1024], ((0, _NTAB - 977), (0, 0)))
    fine = embeddings[:_NTAB]
    tabs = jnp.concatenate([
        coarse[:, 0::2], coarse[:, 1::2], fine[:, 0::2], fine[:, 1::2]])
    out = _encode(t.reshape(-1), tabs.reshape(-1))
    return out.reshape(_N, _SP, _DP)[:, :_S, :_D]


# speedup vs baseline: 1.0558x; 1.0558x over previous
import jax
import jax.numpy as jnp
from jax import lax
from jax.experimental import pallas as pl
from jax.experimental.pallas import tpu as pltpu
from jax.experimental.pallas import tpu_sc as plsc

_INFO = plsc.get_sparse_core_info()
_NC, _NS = _INFO.num_cores, _INFO.num_subcores
_NW = _NC * _NS

_N = 16384
_S = 50
_D = 32
_NF = _D // 2
_NTAB = 1024
_SP = 56
_DP = 128
_IPW = _N // _NW
_ICH = 4
_NSLAB = _IPW // _ICH
_RSLAB = _ICH * _S
_IDXCH = 1600
_SPI = _IDXCH // _RSLAB
_NIDX = _IPW * _S // _IDXCH
_UNROLL = 8
_ROWW = _S * _DP
_SLABW = _SP * _DP

_OFF_SA = 0
_OFF_CA = _NTAB * _NF
_OFF_SB = 2 * _NTAB * _NF
_OFF_CB = 3 * _NTAB * _NF


def _body(t_hbm, tabs_hbm, out_hbm, tabs, idx_v, slab_v, sem_i, sem_o):
    wid = lax.axis_index("s") * _NC + lax.axis_index("c")
    base = wid * _IPW * _S
    ibase = wid * _IPW

    pltpu.sync_copy(tabs_hbm, tabs)

    def idx_load(ic, b):
        pltpu.async_copy(t_hbm.at[pl.ds(base + ic * _IDXCH, _IDXCH)],
                         idx_v[b].at[pl.ds(0, _IDXCH)], sem_i[b])

    def wait_idx(ic, b):
        pltpu.make_async_copy(t_hbm.at[pl.ds(base + ic * _IDXCH, _IDXCH)],
                              idx_v[b].at[pl.ds(0, _IDXCH)], sem_i[b]).wait()

    def store(g, b):
        row0 = (ibase + g * _ICH) * _SLABW
        for ii in range(_ICH):
            pltpu.async_copy(
                slab_v[b].at[pl.ds(ii * _SLABW, _ROWW)],
                out_hbm.at[pl.ds(row0 + ii * _SLABW, _ROWW)], sem_o[b])

    def wait_store(g, b):
        row0 = (ibase + g * _ICH) * _SLABW
        for ii in range(_ICH):
            pltpu.make_async_copy(
                slab_v[b].at[pl.ds(ii * _SLABW, _ROWW)],
                out_hbm.at[pl.ds(row0 + ii * _SLABW, _ROWW)],
                sem_o[b]).wait()

    evens = 2 * lax.iota(jnp.int32, 16)
    odds = evens + 1
    lane = lax.iota(jnp.int32, 16)

    def compute(ib, s, b):
        @plsc.parallel_loop(0, _RSLAB // _UNROLL, unroll=2)
        def _(g):
            tvec = idx_v[ib][pl.ds(s * _RSLAB + g * _UNROLL, 16)]
            a16v = (tvec >> 10) * _NF
            b16v = (tvec & 1023) * _NF
            rrv = g * _UNROLL + lane
            iiv = rrv // _S
            posv = (iiv * (_SP - _S) + rrv) * _DP
            for u in range(_UNROLL):
                a16 = a16v[u]
                b16 = b16v[u]
                s_a = tabs[pl.ds(_OFF_SA + a16, _NF)]
                c_a = tabs[pl.ds(_OFF_CA + a16, _NF)]
                s_b = tabs[pl.ds(_OFF_SB + b16, _NF)]
                c_b = tabs[pl.ds(_OFF_CB + b16, _NF)]
                sv = s_a * c_b + c_a * s_b
                cv = c_a * c_b - s_a * s_b
                pos = jnp.full((16,), posv[u], jnp.int32)
                plsc.store_scatter(slab_v[b], [pos + evens], sv)
                plsc.store_scatter(slab_v[b], [pos + odds], cv)

    idx_load(0, 0)

    def slab_step(ic, ib, s, b, first_two):
        g = ic * _SPI + s
        if not first_two:
            wait_store(g - 2, b)
        compute(ib, s, b)
        store(g, b)

    def slab_pairs(ic, ib, lo):
        @pl.loop(lo, _SPI // 2)
        def _(sp):
            for db in range(2):
                slab_step(ic, ib, sp * 2 + db, db, first_two=False)

    def idx_chunk(ic, prefetch, first=False):
        ib = ic % 2
        wait_idx(ic, ib)
        if prefetch:
            idx_load(ic + 1, (ib + 1) % 2)
        if first:
            slab_step(ic, ib, 0, 0, first_two=True)
            slab_step(ic, ib, 1, 1, first_two=True)
            slab_pairs(ic, ib, 1)
        else:
            slab_pairs(ic, ib, 0)

    idx_chunk(0, prefetch=True, first=True)

    @pl.loop(0, (_NIDX - 2) // 2)
    def _(k):
        for db in range(2):
            ic = 1 + 2 * k + db
            ib = (1 + db) % 2
            wait_idx(ic, ib)
            idx_load(ic + 1, (ib + 1) % 2)
            slab_pairs(ic, ib, 0)

    idx_chunk(_NIDX - 1, prefetch=False)

    wait_store(_NSLAB - 2, 0)
    wait_store(_NSLAB - 1, 1)


@jax.jit
def _encode(t_flat, tabs_flat):
    mesh = plsc.VectorSubcoreMesh(core_axis_name="c", subcore_axis_name="s")
    k = pl.kernel(
        _body,
        out_type=jax.ShapeDtypeStruct((_N * _SP * _DP,), jnp.float32),
        mesh=mesh,
        scratch_types=[
            pltpu.VMEM((4 * _NTAB * _NF,), jnp.float32),
            tuple(pltpu.VMEM((_IDXCH + 16,), jnp.int32) for _ in range(2)),
            tuple(pltpu.VMEM((_ICH * _SLABW,), jnp.float32)
                  for _ in range(2)),
            tuple(pltpu.SemaphoreType.DMA for _ in range(2)),
            tuple(pltpu.SemaphoreType.DMA for _ in range(2)),
        ],
        compiler_params=pltpu.CompilerParams(use_tc_tiling_on_sc=False,
                                             needs_layout_passes=False),
    )
    return k(t_flat, tabs_flat)


def kernel(t, embeddings):
    coarse_idx = jnp.arange(0, _NTAB, dtype=jnp.int32) * 1024
    coarse_idx = jnp.minimum(coarse_idx, 999424)
    coarse = jnp.take(embeddings, coarse_idx, axis=0)
    fine = embeddings[:_NTAB]
    tabs = jnp.concatenate([
        coarse[:, 0::2], coarse[:, 1::2], fine[:, 0::2], fine[:, 1::2]])
    out = _encode(t.reshape(-1), tabs.reshape(-1))
    return out.reshape(_N, _SP, _DP)[:, :_S, :_D]
